# baseline (speedup 1.0000x reference)
import functools

import jax
import jax.numpy as jnp
from jax import lax
from jax.experimental import pallas as pl
from jax.experimental.pallas import tpu as pltpu
from jax.experimental.pallas import tpu_sc as plsc

_NUM_CORES = 2
_NUM_SUBCORES = 16
_NW = _NUM_CORES * _NUM_SUBCORES


def _chunk_starts(lo, hi):
  starts = []
  b = lo // 128
  while b * 128 < hi:
    s0, s1 = max(lo, b * 128), min(hi, (b + 1) * 128)
    seg = list(range(s0, s1 - 16, 16))
    seg.append(s1 - 16)
    starts += seg
    b += 1
  return starts


def _sc_build_pos(dims, z_lo, nz, row_w, col_w, hei_w):
  B, Z, C, X, Y = dims
  CH = col_w.shape[1]
  CH2 = 2 * CH
  CHZ = C - CH2

  mesh = plsc.VectorSubcoreMesh(
      core_axis_name="c", subcore_axis_name="s",
      num_cores=_NUM_CORES, num_subcores=_NUM_SUBCORES)

  @functools.partial(
      pl.kernel,
      out_type=jax.ShapeDtypeStruct((nz, X, Y, C), jnp.float32),
      mesh=mesh,
      scratch_types=[
          pltpu.VMEM((row_w.size,), jnp.float32),
          pltpu.VMEM((col_w.size,), jnp.float32),
          pltpu.VMEM((hei_w.size,), jnp.float32),
          pltpu.VMEM((nz * Y, C), jnp.float32),
          pltpu.SemaphoreType.DMA,
      ],
      compiler_params=pltpu.CompilerParams(needs_layout_passes=False),
  )
  def kern(row_hbm, col_hbm, hei_hbm, pos_hbm, roww_v, colw_v, heiw_v,
           slabs_v, sem):
    cid = lax.axis_index("c")
    sid = lax.axis_index("s")
    wid = sid * _NUM_CORES + cid

    pltpu.sync_copy(row_hbm, roww_v)
    pltpu.sync_copy(col_hbm, colw_v)
    pltpu.sync_copy(hei_hbm, heiw_v)

    col_s = _chunk_starts(0, CH)
    row_s = _chunk_starts(CH, CH2)
    hei_s = _chunk_starts(CH2, C)

    rw = [roww_v[pl.ds(wid * CH + (s - CH), 16)] for s in row_s]

    def per_z(z, _):
      hz = [heiw_v[pl.ds((z_lo + z) * CH + (s - CH2), 16)] for s in hei_s]

      def per_y(y, _):
        r = z * Y + y
        for s in col_s:
          slabs_v[r, pl.ds(s, 16)] = colw_v[pl.ds(y * CH + s, 16)]
        for v, s in zip(rw, row_s):
          slabs_v[r, pl.ds(s, 16)] = v
        for v, s in zip(hz, hei_s):
          slabs_v[r, pl.ds(s, 16)] = v
        return 0

      lax.fori_loop(0, Y, per_y, 0)
      return 0

    lax.fori_loop(0, nz, per_z, 0)

    def issue(z, _):
      pltpu.async_copy(
          slabs_v.at[pl.ds(z * Y, Y)], pos_hbm.at[z, wid], sem)
      return 0

    lax.fori_loop(0, nz, issue, 0)

    def drain(z, _):
      pltpu.make_async_copy(
          slabs_v.at[pl.ds(0, Y)], pos_hbm.at[0, wid], sem).wait()
      return 0

    lax.fori_loop(0, nz, drain, 0)

  return kern(row_w.reshape(-1), col_w.reshape(-1), hei_w.reshape(-1))


def _tc_broadcast(dims, z_lo, nz, pos, prev=None):
  B, Z, C, X, Y = dims

  def body(*refs):
    pos_hbm, out_hbm, pos_v, sem_in, sem_out = refs[:1] + refs[-4:]
    cp = pltpu.make_async_copy(pos_hbm, pos_v, sem_in)
    cp.start()
    cp.wait()

    def issue(b, _):
      pltpu.make_async_copy(
          pos_v, out_hbm.at[b, pl.ds(z_lo, nz)], sem_out).start()
      return 0

    lax.fori_loop(0, B, issue, 0, unroll=4)

    def drain(b, _):
      pltpu.make_async_copy(
          pos_v, out_hbm.at[0, pl.ds(z_lo, nz)], sem_out).wait()
      return 0

    lax.fori_loop(0, B, drain, 0)

  args = (pos,) if prev is None else (pos, prev)
  return pl.pallas_call(
      body,
      out_shape=jax.ShapeDtypeStruct((B, Z, X, Y, C), jnp.float32),
      in_specs=[pl.BlockSpec(memory_space=pl.ANY)] * len(args),
      out_specs=pl.BlockSpec(memory_space=pl.ANY),
      input_output_aliases={} if prev is None else {1: 0},
      scratch_shapes=[
          pltpu.VMEM((nz, X, Y, C), jnp.float32),
          pltpu.SemaphoreType.DMA,
          pltpu.SemaphoreType.DMA,
      ],
  )(*args)


_Z_SPLIT = 2


@functools.partial(jax.jit, static_argnums=(0,))
def _pos_embed(dims, row_w, col_w, hei_w):
  B, Z, C, X, Y = dims
  z1 = _Z_SPLIT
  pos_a = _sc_build_pos(dims, 0, z1, row_w, col_w, hei_w)
  pos_b = _sc_build_pos(dims, z1, Z - z1, row_w, col_w, hei_w)
  out = _tc_broadcast(dims, 0, z1, pos_a)
  out = _tc_broadcast(dims, z1, Z - z1, pos_b, prev=out)
  return jnp.transpose(out, (0, 1, 4, 2, 3))


def kernel(tensor, row_w, col_w, hei_w):
  B, Z, C, X, Y = tensor.shape
  assert X == _NW and Y == X and C > 2 * row_w.shape[1]
  return _pos_embed((B, Z, C, X, Y), row_w, col_w, hei_w)


# speedup vs baseline: 1.0779x; 1.0779x over previous
import functools

import jax
import jax.numpy as jnp
from jax import lax
from jax.experimental import pallas as pl
from jax.experimental.pallas import tpu as pltpu
from jax.experimental.pallas import tpu_sc as plsc

_NUM_CORES = 2
_NUM_SUBCORES = 16
_NW = _NUM_CORES * _NUM_SUBCORES


def _chunk_starts(lo, hi):
  starts = []
  b = lo // 128
  while b * 128 < hi:
    s0, s1 = max(lo, b * 128), min(hi, (b + 1) * 128)
    seg = list(range(s0, s1 - 16, 16))
    seg.append(s1 - 16)
    starts += seg
    b += 1
  return starts


def _sc_build_pos(dims, z_lo, nz, row_w, col_w, hei_w):
  B, Z, C, X, Y = dims
  CH = col_w.shape[1]
  CH2 = 2 * CH
  CHZ = C - CH2

  mesh = plsc.VectorSubcoreMesh(
      core_axis_name="c", subcore_axis_name="s",
      num_cores=_NUM_CORES, num_subcores=_NUM_SUBCORES)

  @functools.partial(
      pl.kernel,
      out_type=jax.ShapeDtypeStruct((nz, X, Y, C), jnp.float32),
      mesh=mesh,
      scratch_types=[
          pltpu.VMEM((row_w.size,), jnp.float32),
          pltpu.VMEM((col_w.size,), jnp.float32),
          pltpu.VMEM((hei_w.size,), jnp.float32),
          pltpu.VMEM((nz * Y, C), jnp.float32),
          pltpu.SemaphoreType.DMA,
      ],
      compiler_params=pltpu.CompilerParams(needs_layout_passes=False),
  )
  def kern(row_hbm, col_hbm, hei_hbm, pos_hbm, roww_v, colw_v, heiw_v,
           slabs_v, sem):
    cid = lax.axis_index("c")
    sid = lax.axis_index("s")
    wid = sid * _NUM_CORES + cid

    pltpu.sync_copy(row_hbm, roww_v)
    pltpu.sync_copy(col_hbm, colw_v)
    pltpu.sync_copy(hei_hbm, heiw_v)

    col_s = _chunk_starts(0, CH)
    row_s = _chunk_starts(CH, CH2)
    hei_s = _chunk_starts(CH2, C)

    rw = [roww_v[pl.ds(wid * CH + (s - CH), 16)] for s in row_s]

    def per_z(z, _):
      hz = [heiw_v[pl.ds((z_lo + z) * CH + (s - CH2), 16)] for s in hei_s]

      def per_y(y, _):
        r = z * Y + y
        for s in col_s:
          slabs_v[r, pl.ds(s, 16)] = colw_v[pl.ds(y * CH + s, 16)]
        for v, s in zip(rw, row_s):
          slabs_v[r, pl.ds(s, 16)] = v
        for v, s in zip(hz, hei_s):
          slabs_v[r, pl.ds(s, 16)] = v
        return 0

      lax.fori_loop(0, Y, per_y, 0)
      return 0

    lax.fori_loop(0, nz, per_z, 0)

    def issue(z, _):
      pltpu.async_copy(
          slabs_v.at[pl.ds(z * Y, Y)], pos_hbm.at[z, wid], sem)
      return 0

    lax.fori_loop(0, nz, issue, 0)

    def drain(z, _):
      pltpu.make_async_copy(
          slabs_v.at[pl.ds(0, Y)], pos_hbm.at[0, wid], sem).wait()
      return 0

    lax.fori_loop(0, nz, drain, 0)

  return kern(row_w.reshape(-1), col_w.reshape(-1), hei_w.reshape(-1))


def _tc_broadcast(dims, z_lo, nz, pos, prev=None):
  B, Z, C, X, Y = dims

  def body(*refs):
    pos_hbm, out_hbm, pos_v, sem_in, sem_out = refs[:1] + refs[-4:]
    cp = pltpu.make_async_copy(pos_hbm, pos_v, sem_in)
    cp.start()
    cp.wait()

    def issue(b, _):
      pltpu.make_async_copy(
          pos_v, out_hbm.at[b, pl.ds(z_lo, nz)], sem_out).start()
      return 0

    lax.fori_loop(0, B, issue, 0, unroll=4)

    def drain(b, _):
      pltpu.make_async_copy(
          pos_v, out_hbm.at[0, pl.ds(z_lo, nz)], sem_out).wait()
      return 0

    lax.fori_loop(0, B, drain, 0)

  args = (pos,) if prev is None else (pos, prev)
  return pl.pallas_call(
      body,
      out_shape=jax.ShapeDtypeStruct((B, Z, X, Y, C), jnp.float32),
      in_specs=[pl.BlockSpec(memory_space=pl.ANY)] * len(args),
      out_specs=pl.BlockSpec(memory_space=pl.ANY),
      input_output_aliases={} if prev is None else {1: 0},
      scratch_shapes=[
          pltpu.VMEM((nz, X, Y, C), jnp.float32),
          pltpu.SemaphoreType.DMA,
          pltpu.SemaphoreType.DMA,
      ],
  )(*args)


@functools.partial(jax.jit, static_argnums=(0,))
def _pos_embed(dims, row_w, col_w, hei_w):
  B, Z, C, X, Y = dims
  pos = _sc_build_pos(dims, 0, Z, row_w, col_w, hei_w)
  out = _tc_broadcast(dims, 0, Z, pos)
  return jnp.transpose(out, (0, 1, 4, 2, 3))


def kernel(tensor, row_w, col_w, hei_w):
  B, Z, C, X, Y = tensor.shape
  assert X == _NW and Y == X and C > 2 * row_w.shape[1]
  return _pos_embed((B, Z, C, X, Y), row_w, col_w, hei_w)


# speedup vs baseline: 1.0827x; 1.0045x over previous
import functools

import jax
import jax.numpy as jnp
from jax import lax
from jax.experimental import pallas as pl
from jax.experimental.pallas import tpu as pltpu
from jax.experimental.pallas import tpu_sc as plsc

_NUM_CORES = 2
_NUM_SUBCORES = 16
_NW = _NUM_CORES * _NUM_SUBCORES


def _chunk_starts(lo, hi):
  starts = []
  b = lo // 128
  while b * 128 < hi:
    s0, s1 = max(lo, b * 128), min(hi, (b + 1) * 128)
    seg = list(range(s0, s1 - 16, 16))
    seg.append(s1 - 16)
    starts += seg
    b += 1
  return starts


def _sc_build_pos(dims, z_lo, nz, row_w, col_w, hei_w):
  B, Z, C, X, Y = dims
  CH = col_w.shape[1]
  CH2 = 2 * CH
  CHZ = C - CH2

  mesh = plsc.VectorSubcoreMesh(
      core_axis_name="c", subcore_axis_name="s",
      num_cores=_NUM_CORES, num_subcores=_NUM_SUBCORES)

  @functools.partial(
      pl.kernel,
      out_type=jax.ShapeDtypeStruct((nz, X, Y, C), jnp.float32),
      mesh=mesh,
      scratch_types=[
          pltpu.VMEM((row_w.size,), jnp.float32),
          pltpu.VMEM((col_w.size,), jnp.float32),
          pltpu.VMEM((hei_w.size,), jnp.float32),
          pltpu.VMEM((nz * Y, C), jnp.float32),
          pltpu.SemaphoreType.DMA,
      ],
      compiler_params=pltpu.CompilerParams(needs_layout_passes=False),
  )
  def kern(row_hbm, col_hbm, hei_hbm, pos_hbm, roww_v, colw_v, heiw_v,
           slabs_v, sem):
    cid = lax.axis_index("c")
    sid = lax.axis_index("s")
    wid = sid * _NUM_CORES + cid

    pltpu.async_copy(row_hbm, roww_v, sem)
    pltpu.async_copy(col_hbm, colw_v, sem)
    pltpu.async_copy(hei_hbm, heiw_v, sem)
    pltpu.make_async_copy(row_hbm, roww_v, sem).wait()
    pltpu.make_async_copy(col_hbm, colw_v, sem).wait()
    pltpu.make_async_copy(hei_hbm, heiw_v, sem).wait()

    col_s = _chunk_starts(0, CH)
    row_s = _chunk_starts(CH, CH2)
    hei_s = _chunk_starts(CH2, C)

    rw = [roww_v[pl.ds(wid * CH + (s - CH), 16)] for s in row_s]

    def per_z(z, _):
      hz = [heiw_v[pl.ds((z_lo + z) * CH + (s - CH2), 16)] for s in hei_s]

      def per_y(y, _):
        r = z * Y + y
        for s in col_s:
          slabs_v[r, pl.ds(s, 16)] = colw_v[pl.ds(y * CH + s, 16)]
        for v, s in zip(rw, row_s):
          slabs_v[r, pl.ds(s, 16)] = v
        for v, s in zip(hz, hei_s):
          slabs_v[r, pl.ds(s, 16)] = v
        return 0

      lax.fori_loop(0, Y, per_y, 0, unroll=4)
      return 0

    lax.fori_loop(0, nz, per_z, 0)

    def issue(z, _):
      pltpu.async_copy(
          slabs_v.at[pl.ds(z * Y, Y)], pos_hbm.at[z, wid], sem)
      return 0

    lax.fori_loop(0, nz, issue, 0)

    def drain(z, _):
      pltpu.make_async_copy(
          slabs_v.at[pl.ds(0, Y)], pos_hbm.at[0, wid], sem).wait()
      return 0

    lax.fori_loop(0, nz, drain, 0)

  return kern(row_w.reshape(-1), col_w.reshape(-1), hei_w.reshape(-1))


def _tc_broadcast(dims, z_lo, nz, pos, prev=None):
  B, Z, C, X, Y = dims

  def body(*refs):
    pos_hbm, out_hbm, pos_v, sem_in, sem_out = refs[:1] + refs[-4:]
    cp = pltpu.make_async_copy(pos_hbm, pos_v, sem_in)
    cp.start()
    cp.wait()

    def issue(b, _):
      pltpu.make_async_copy(
          pos_v, out_hbm.at[b, pl.ds(z_lo, nz)], sem_out).start()
      return 0

    lax.fori_loop(0, B, issue, 0, unroll=4)

    def drain(b, _):
      pltpu.make_async_copy(
          pos_v, out_hbm.at[0, pl.ds(z_lo, nz)], sem_out).wait()
      return 0

    lax.fori_loop(0, B, drain, 0)

  args = (pos,) if prev is None else (pos, prev)
  return pl.pallas_call(
      body,
      out_shape=jax.ShapeDtypeStruct((B, Z, X, Y, C), jnp.float32),
      in_specs=[pl.BlockSpec(memory_space=pl.ANY)] * len(args),
      out_specs=pl.BlockSpec(memory_space=pl.ANY),
      input_output_aliases={} if prev is None else {1: 0},
      scratch_shapes=[
          pltpu.VMEM((nz, X, Y, C), jnp.float32),
          pltpu.SemaphoreType.DMA,
          pltpu.SemaphoreType.DMA,
      ],
  )(*args)


@functools.partial(jax.jit, static_argnums=(0,))
def _pos_embed(dims, row_w, col_w, hei_w):
  B, Z, C, X, Y = dims
  pos = _sc_build_pos(dims, 0, Z, row_w, col_w, hei_w)
  out = _tc_broadcast(dims, 0, Z, pos)
  return jnp.transpose(out, (0, 1, 4, 2, 3))


def kernel(tensor, row_w, col_w, hei_w):
  B, Z, C, X, Y = tensor.shape
  assert X == _NW and Y == X and C > 2 * row_w.shape[1]
  return _pos_embed((B, Z, C, X, Y), row_w, col_w, hei_w)
